# baseline (speedup 1.0000x reference)
import functools

import jax
import jax.numpy as jnp
from jax import lax
from jax.experimental import pallas as pl
from jax.experimental.pallas import tpu as pltpu
from jax.experimental.pallas import tpu_sc as plsc

N = 10000
E = 320000
D = 128
NT = 16
RPT = N // NT
RSTAGE = 25

CHUNK = 125
NG = 4
GC = 40
NROW = N

DCHUNK = 80
DNCHUNK = (E // NT) // DCHUNK
MDEG = 10240
DLANE = 16

_mesh = plsc.VectorSubcoreMesh(core_axis_name="c", subcore_axis_name="s")
_sc_params = pltpu.CompilerParams(use_tc_tiling_on_sc=False)


def _deg_body(dstL_hbm, dstN_hbm, ones_hbm, zeros_hbm, out_hbm,
              deg_sh, idx_v, ones_v, stage_v):
    c = lax.axis_index("c")
    s = lax.axis_index("s")
    pltpu.sync_copy(ones_hbm, ones_v)
    pltpu.sync_copy(zeros_hbm, stage_v)
    pltpu.sync_copy(stage_v, deg_sh.at[pl.ds(s * 640, 640), :])

    @pl.when(c == 0)
    def _():
        pltpu.sync_copy(dstL_hbm.at[s], idx_v)

    @pl.when(c == 1)
    def _():
        pltpu.sync_copy(dstN_hbm.at[s], idx_v)

    plsc.subcore_barrier()

    def body(j, carry):
        pltpu.sync_copy(ones_v, deg_sh.at[idx_v.at[j]], add=True)
        return carry

    lax.fori_loop(0, DNCHUNK, body, 0)
    plsc.subcore_barrier()
    pltpu.sync_copy(deg_sh.at[pl.ds(s * 640, 640), :], stage_v)
    pltpu.sync_copy(stage_v, out_hbm.at[c, pl.ds(s * 640, 640), :])


_deg_kernel = pl.kernel(
    _deg_body,
    out_type=jax.ShapeDtypeStruct((2, MDEG, DLANE), jnp.float32),
    mesh=_mesh,
    scratch_types=[
        pltpu.VMEM_SHARED((MDEG, DLANE), jnp.float32),
        pltpu.VMEM((DNCHUNK, DCHUNK), jnp.int32),
        pltpu.VMEM((DCHUNK, DLANE), jnp.float32),
        pltpu.VMEM((640, DLANE), jnp.float32),
    ],
    compiler_params=_sc_params,
)


CH = 64


def _conv_body(yL_hbm, yN_hbm, sL_hbm, dL_hbm, sN_hbm, dN_hbm, out_hbm,
               src_sh, acc_sh, sidx_v, didx_v, rows_v, stage_v):
    c = lax.axis_index("c")
    s = lax.axis_index("s")
    base = s * RPT

    for h in range(D // CH):
        col = pl.ds(h * CH, CH)

        def init(y_hbm):
            def b(t, carry):
                sl = pl.ds(base + t * RSTAGE, RSTAGE)
                pltpu.sync_copy(y_hbm.at[sl, col], stage_v)
                pltpu.sync_copy(stage_v, src_sh.at[sl, :])
                pltpu.sync_copy(stage_v, acc_sh.at[sl, :])
                return carry
            lax.fori_loop(0, RPT // RSTAGE, b, 0)

        @pl.when(c == 0)
        def _():
            init(yL_hbm)

        @pl.when(c == 1)
        def _():
            init(yN_hbm)

        plsc.subcore_barrier()

        def edges(s4, d4):
            def gbody(g, carry):
                pltpu.sync_copy(s4.at[s, g], sidx_v)
                pltpu.sync_copy(d4.at[s, g], didx_v)

                def cbody(j, carry2):
                    pltpu.sync_copy(src_sh.at[sidx_v.at[j]], rows_v)
                    pltpu.sync_copy(rows_v, acc_sh.at[didx_v.at[j]], add=True)
                    return carry2

                lax.fori_loop(0, GC, cbody, 0)
                return carry

            lax.fori_loop(0, NG, gbody, 0)

        @pl.when(c == 0)
        def _():
            edges(sL_hbm, dL_hbm)

        @pl.when(c == 1)
        def _():
            edges(sN_hbm, dN_hbm)

        plsc.subcore_barrier()

        def wb(t, carry):
            sl = pl.ds(base + t * RSTAGE, RSTAGE)
            pltpu.sync_copy(acc_sh.at[sl, :], stage_v)
            pltpu.sync_copy(stage_v, out_hbm.at[c, sl, col])
            return carry

        lax.fori_loop(0, RPT // RSTAGE, wb, 0)


_conv_kernel = pl.kernel(
    _conv_body,
    out_type=jax.ShapeDtypeStruct((2, N, D), jnp.float32),
    mesh=_mesh,
    scratch_types=[
        pltpu.VMEM_SHARED((NROW, CH), jnp.float32),
        pltpu.VMEM_SHARED((NROW, CH), jnp.float32),
        pltpu.VMEM((GC, CHUNK), jnp.int32),
        pltpu.VMEM((GC, CHUNK), jnp.int32),
        pltpu.VMEM((CHUNK, CH), jnp.float32),
        pltpu.VMEM((RSTAGE, CH), jnp.float32),
    ],
    compiler_params=_sc_params,
)


GRID = 10
RB = N // GRID

_row = pl.BlockSpec((RB, D), lambda i: (i, 0))
_col = pl.BlockSpec((RB, 1), lambda i: (i, 0))
_full = pl.BlockSpec((D, D), lambda i: (0, 0))
_vec = pl.BlockSpec((1, D), lambda i: (0, 0))


def _mm_body(x_ref, w0_ref, w1_ref, dl_ref, dn_ref, yl_ref, yn_ref):
    dinv_l = lax.rsqrt(dl_ref[...] + 1.0)
    dinv_n = lax.rsqrt(dn_ref[...] + 1.0)
    xb = x_ref[...]
    yl_ref[...] = jnp.dot(xb, w0_ref[...], preferred_element_type=jnp.float32) * dinv_l
    yn_ref[...] = jnp.dot(xb, w1_ref[...], preferred_element_type=jnp.float32) * dinv_n


_mm = pl.pallas_call(
    _mm_body,
    grid=(GRID,),
    in_specs=[_row, _full, _full, _col, _col],
    out_specs=[_row, _row],
    out_shape=[jax.ShapeDtypeStruct((N, D), jnp.float32)] * 2,
)


def _combine_stats_body(al_ref, an_ref, dl_ref, dn_ref, b_ref,
                        h_ref, st_ref):
    i = pl.program_id(0)
    dinv_l = lax.rsqrt(dl_ref[...] + 1.0)
    dinv_n = lax.rsqrt(dn_ref[...] + 1.0)
    h = al_ref[...] * dinv_l + 0.5 * (an_ref[...] * dinv_n) + b_ref[...]
    h_ref[...] = h
    st = jnp.concatenate(
        [jnp.sum(h, axis=0, keepdims=True),
         jnp.sum(h * h, axis=0, keepdims=True)], axis=0)

    @pl.when(i == 0)
    def _():
        st_ref[...] = st

    @pl.when(i > 0)
    def _():
        st_ref[...] += st


_combine_stats = pl.pallas_call(
    _combine_stats_body,
    grid=(GRID,),
    in_specs=[_row, _row, _col, _col, _vec],
    out_specs=[_row, pl.BlockSpec((2, D), lambda i: (0, 0))],
    out_shape=[jax.ShapeDtypeStruct((N, D), jnp.float32),
               jax.ShapeDtypeStruct((2, D), jnp.float32)],
)


def _bn_mm_body(h_ref, st_ref, g_ref, be_ref, w0_ref, w1_ref, dl_ref, dn_ref,
                yl_ref, yn_ref):
    mean = st_ref[0:1, :] * (1.0 / N)
    var = st_ref[1:2, :] * (1.0 / N) - mean * mean
    inv = lax.rsqrt(var + 1e-5)
    h = jnp.maximum((h_ref[...] - mean) * inv * g_ref[...] + be_ref[...], 0.0)
    dinv_l = lax.rsqrt(dl_ref[...] + 1.0)
    dinv_n = lax.rsqrt(dn_ref[...] + 1.0)
    yl_ref[...] = jnp.dot(h, w0_ref[...], preferred_element_type=jnp.float32) * dinv_l
    yn_ref[...] = jnp.dot(h, w1_ref[...], preferred_element_type=jnp.float32) * dinv_n


_bn_mm = pl.pallas_call(
    _bn_mm_body,
    grid=(GRID,),
    in_specs=[_row, pl.BlockSpec((2, D), lambda i: (0, 0)), _vec, _vec,
              _full, _full, _col, _col],
    out_specs=[_row, _row],
    out_shape=[jax.ShapeDtypeStruct((N, D), jnp.float32)] * 2,
)


def _final_body(al_ref, an_ref, dl_ref, dn_ref, b_ref, o_ref):
    dinv_l = lax.rsqrt(dl_ref[...] + 1.0)
    dinv_n = lax.rsqrt(dn_ref[...] + 1.0)
    o_ref[...] = al_ref[...] * dinv_l + 0.5 * (an_ref[...] * dinv_n) + b_ref[...]


_final = pl.pallas_call(
    _final_body,
    grid=(GRID,),
    in_specs=[_row, _row, _col, _col, _vec],
    out_specs=_row,
    out_shape=jax.ShapeDtypeStruct((N, D), jnp.float32),
)


def kernel(x, adj_low, adj_high, adj_nd_low, adj_nd_high,
           W0, b0, Whi0, bhi0, W1, b1, Whi1, bhi1, gamma, beta):
    adj_low = adj_low.astype(jnp.int32)
    adj_nd = adj_nd_low.astype(jnp.int32)

    def pad4(src, dst):
        s4 = src.reshape(NT, NG, GC, CHUNK)
        d4 = dst.reshape(NT, NG, GC, CHUNK)
        return s4, d4

    sL4, dL4 = pad4(adj_low[0], adj_low[1])
    sN4, dN4 = pad4(adj_nd[0], adj_nd[1])
    dL3 = adj_low[1].reshape(NT, DNCHUNK, DCHUNK)
    dN3 = adj_nd[1].reshape(NT, DNCHUNK, DCHUNK)

    ones = jnp.ones((DCHUNK, DLANE), jnp.float32)
    zeros = jnp.zeros((640, DLANE), jnp.float32)
    deg = _deg_kernel(dL3, dN3, ones, zeros)
    deg_l = deg[0, :N, 0:1]
    deg_n = deg[1, :N, 0:1]

    bias0 = (b0 + 0.5 * bhi0).reshape(1, D)
    bias1 = (b1 + 0.5 * bhi1).reshape(1, D)

    y0_l, y0_n = _mm(x, W0, Whi0, deg_l, deg_n)
    acc0 = _conv_kernel(y0_l, y0_n, sL4, dL4, sN4, dN4)
    h_pre, stats = _combine_stats(acc0[0], acc0[1], deg_l, deg_n, bias0)
    y1_l, y1_n = _bn_mm(h_pre, stats, gamma.reshape(1, D), beta.reshape(1, D),
                        W1, Whi1, deg_l, deg_n)
    acc1 = _conv_kernel(y1_l, y1_n, sL4, dL4, sN4, dN4)
    return _final(acc1[0], acc1[1], deg_l, deg_n, bias1)


# speedup vs baseline: 1.2108x; 1.2108x over previous
import functools

import jax
import jax.numpy as jnp
from jax import lax
from jax.experimental import pallas as pl
from jax.experimental.pallas import tpu as pltpu
from jax.experimental.pallas import tpu_sc as plsc

N = 10000
E = 320000
D = 128
NT = 16
RPT = N // NT
RSTAGE = 25

CHUNK = 125
NG = 4
GC = 40
NROW = N

DCHUNK = 125
DNCHUNK = (E // NT) // DCHUNK
MDEG = 10240
DLANE = 16

_mesh = plsc.VectorSubcoreMesh(core_axis_name="c", subcore_axis_name="s")
_sc_params = pltpu.CompilerParams(use_tc_tiling_on_sc=False)


def _deg_body(dstL_hbm, dstN_hbm, ones_hbm, zeros_hbm, out_hbm,
              deg_sh, idx_v, ones_v, stage_v):
    c = lax.axis_index("c")
    s = lax.axis_index("s")
    pltpu.sync_copy(ones_hbm, ones_v)
    pltpu.sync_copy(zeros_hbm, stage_v)
    pltpu.sync_copy(stage_v, deg_sh.at[pl.ds(s * 640, 640), :])

    @pl.when(c == 0)
    def _():
        pltpu.sync_copy(dstL_hbm.at[s], idx_v)

    @pl.when(c == 1)
    def _():
        pltpu.sync_copy(dstN_hbm.at[s], idx_v)

    plsc.subcore_barrier()

    def body(j, carry):
        pltpu.sync_copy(ones_v, deg_sh.at[idx_v.at[j]], add=True)
        return carry

    lax.fori_loop(0, DNCHUNK, body, 0)
    plsc.subcore_barrier()
    pltpu.sync_copy(deg_sh.at[pl.ds(s * 640, 640), :], stage_v)
    pltpu.sync_copy(stage_v, out_hbm.at[c, pl.ds(s * 640, 640), :])


_deg_kernel = pl.kernel(
    _deg_body,
    out_type=jax.ShapeDtypeStruct((2, MDEG, DLANE), jnp.float32),
    mesh=_mesh,
    scratch_types=[
        pltpu.VMEM_SHARED((MDEG, DLANE), jnp.float32),
        pltpu.VMEM((DNCHUNK, DCHUNK), jnp.int32),
        pltpu.VMEM((DCHUNK, DLANE), jnp.float32),
        pltpu.VMEM((640, DLANE), jnp.float32),
    ],
    compiler_params=_sc_params,
)


def _conv_body(yL_hbm, yN_hbm, sL_hbm, dL_hbm, sN_hbm, dN_hbm, out_hbm,
               acc_sh, sidx_v, didx_v, rows_v, stage_v):
    c = lax.axis_index("c")
    s = lax.axis_index("s")
    base = s * RPT

    def init(y_hbm):
        def b(t, carry):
            sl = pl.ds(base + t * RSTAGE, RSTAGE)
            pltpu.sync_copy(y_hbm.at[sl, :], stage_v)
            pltpu.sync_copy(stage_v, acc_sh.at[sl, :])
            return carry
        lax.fori_loop(0, RPT // RSTAGE, b, 0)

    @pl.when(c == 0)
    def _():
        init(yL_hbm)

    @pl.when(c == 1)
    def _():
        init(yN_hbm)

    plsc.subcore_barrier()

    def edges(y_hbm, s4, d4):
        def gbody(g, carry):
            pltpu.sync_copy(s4.at[s, g], sidx_v)
            pltpu.sync_copy(d4.at[s, g], didx_v)

            def cbody(j, carry2):
                pltpu.sync_copy(y_hbm.at[sidx_v.at[j]], rows_v)
                pltpu.sync_copy(rows_v, acc_sh.at[didx_v.at[j]], add=True)
                return carry2

            lax.fori_loop(0, GC, cbody, 0)
            return carry

        lax.fori_loop(0, NG, gbody, 0)

    @pl.when(c == 0)
    def _():
        edges(yL_hbm, sL_hbm, dL_hbm)

    @pl.when(c == 1)
    def _():
        edges(yN_hbm, sN_hbm, dN_hbm)

    plsc.subcore_barrier()

    def wb(t, carry):
        sl = pl.ds(base + t * RSTAGE, RSTAGE)
        pltpu.sync_copy(acc_sh.at[sl, :], stage_v)
        pltpu.sync_copy(stage_v, out_hbm.at[c, sl, :])
        return carry

    lax.fori_loop(0, RPT // RSTAGE, wb, 0)


_conv_kernel = pl.kernel(
    _conv_body,
    out_type=jax.ShapeDtypeStruct((2, N, D), jnp.float32),
    mesh=_mesh,
    scratch_types=[
        pltpu.VMEM_SHARED((NROW, D), jnp.float32),
        pltpu.VMEM((GC, CHUNK), jnp.int32),
        pltpu.VMEM((GC, CHUNK), jnp.int32),
        pltpu.VMEM((CHUNK, D), jnp.float32),
        pltpu.VMEM((RSTAGE, D), jnp.float32),
    ],
    compiler_params=_sc_params,
)


GRID = 10
RB = N // GRID

_row = pl.BlockSpec((RB, D), lambda i: (i, 0))
_col = pl.BlockSpec((RB, 1), lambda i: (i, 0))
_full = pl.BlockSpec((D, D), lambda i: (0, 0))
_vec = pl.BlockSpec((1, D), lambda i: (0, 0))


def _mm_body(x_ref, w0_ref, w1_ref, dl_ref, dn_ref, yl_ref, yn_ref):
    dinv_l = lax.rsqrt(dl_ref[...] + 1.0)
    dinv_n = lax.rsqrt(dn_ref[...] + 1.0)
    xb = x_ref[...]
    yl_ref[...] = jnp.dot(xb, w0_ref[...], preferred_element_type=jnp.float32) * dinv_l
    yn_ref[...] = jnp.dot(xb, w1_ref[...], preferred_element_type=jnp.float32) * dinv_n


_mm = pl.pallas_call(
    _mm_body,
    grid=(GRID,),
    in_specs=[_row, _full, _full, _col, _col],
    out_specs=[_row, _row],
    out_shape=[jax.ShapeDtypeStruct((N, D), jnp.float32)] * 2,
)


def _combine_stats_body(al_ref, an_ref, dl_ref, dn_ref, b_ref,
                        h_ref, st_ref):
    i = pl.program_id(0)
    dinv_l = lax.rsqrt(dl_ref[...] + 1.0)
    dinv_n = lax.rsqrt(dn_ref[...] + 1.0)
    h = al_ref[...] * dinv_l + 0.5 * (an_ref[...] * dinv_n) + b_ref[...]
    h_ref[...] = h
    st = jnp.concatenate(
        [jnp.sum(h, axis=0, keepdims=True),
         jnp.sum(h * h, axis=0, keepdims=True)], axis=0)

    @pl.when(i == 0)
    def _():
        st_ref[...] = st

    @pl.when(i > 0)
    def _():
        st_ref[...] += st


_combine_stats = pl.pallas_call(
    _combine_stats_body,
    grid=(GRID,),
    in_specs=[_row, _row, _col, _col, _vec],
    out_specs=[_row, pl.BlockSpec((2, D), lambda i: (0, 0))],
    out_shape=[jax.ShapeDtypeStruct((N, D), jnp.float32),
               jax.ShapeDtypeStruct((2, D), jnp.float32)],
)


def _bn_mm_body(h_ref, st_ref, g_ref, be_ref, w0_ref, w1_ref, dl_ref, dn_ref,
                yl_ref, yn_ref):
    mean = st_ref[0:1, :] * (1.0 / N)
    var = st_ref[1:2, :] * (1.0 / N) - mean * mean
    inv = lax.rsqrt(var + 1e-5)
    h = jnp.maximum((h_ref[...] - mean) * inv * g_ref[...] + be_ref[...], 0.0)
    dinv_l = lax.rsqrt(dl_ref[...] + 1.0)
    dinv_n = lax.rsqrt(dn_ref[...] + 1.0)
    yl_ref[...] = jnp.dot(h, w0_ref[...], preferred_element_type=jnp.float32) * dinv_l
    yn_ref[...] = jnp.dot(h, w1_ref[...], preferred_element_type=jnp.float32) * dinv_n


_bn_mm = pl.pallas_call(
    _bn_mm_body,
    grid=(GRID,),
    in_specs=[_row, pl.BlockSpec((2, D), lambda i: (0, 0)), _vec, _vec,
              _full, _full, _col, _col],
    out_specs=[_row, _row],
    out_shape=[jax.ShapeDtypeStruct((N, D), jnp.float32)] * 2,
)


def _final_body(al_ref, an_ref, dl_ref, dn_ref, b_ref, o_ref):
    dinv_l = lax.rsqrt(dl_ref[...] + 1.0)
    dinv_n = lax.rsqrt(dn_ref[...] + 1.0)
    o_ref[...] = al_ref[...] * dinv_l + 0.5 * (an_ref[...] * dinv_n) + b_ref[...]


_final = pl.pallas_call(
    _final_body,
    grid=(GRID,),
    in_specs=[_row, _row, _col, _col, _vec],
    out_specs=_row,
    out_shape=jax.ShapeDtypeStruct((N, D), jnp.float32),
)


def kernel(x, adj_low, adj_high, adj_nd_low, adj_nd_high,
           W0, b0, Whi0, bhi0, W1, b1, Whi1, bhi1, gamma, beta):
    adj_low = adj_low.astype(jnp.int32)
    adj_nd = adj_nd_low.astype(jnp.int32)

    def pad4(src, dst):
        s4 = src.reshape(NT, NG, GC, CHUNK)
        d4 = dst.reshape(NT, NG, GC, CHUNK)
        return s4, d4

    sL4, dL4 = pad4(adj_low[0], adj_low[1])
    sN4, dN4 = pad4(adj_nd[0], adj_nd[1])
    dL3 = adj_low[1].reshape(NT, DNCHUNK, DCHUNK)
    dN3 = adj_nd[1].reshape(NT, DNCHUNK, DCHUNK)

    ones = jnp.ones((DCHUNK, DLANE), jnp.float32)
    zeros = jnp.zeros((640, DLANE), jnp.float32)
    deg = _deg_kernel(dL3, dN3, ones, zeros)
    deg_l = deg[0, :N, 0:1]
    deg_n = deg[1, :N, 0:1]

    bias0 = (b0 + 0.5 * bhi0).reshape(1, D)
    bias1 = (b1 + 0.5 * bhi1).reshape(1, D)

    y0_l, y0_n = _mm(x, W0, Whi0, deg_l, deg_n)
    acc0 = _conv_kernel(y0_l, y0_n, sL4, dL4, sN4, dN4)
    h_pre, stats = _combine_stats(acc0[0], acc0[1], deg_l, deg_n, bias0)
    y1_l, y1_n = _bn_mm(h_pre, stats, gamma.reshape(1, D), beta.reshape(1, D),
                        W1, Whi1, deg_l, deg_n)
    acc1 = _conv_kernel(y1_l, y1_n, sL4, dL4, sN4, dN4)
    return _final(acc1[0], acc1[1], deg_l, deg_n, bias1)
